# baseline (speedup 1.0000x reference)
import jax
import jax.numpy as jnp
from jax import lax
from jax.experimental import pallas as pl
from jax.experimental.pallas import tpu as pltpu
from jax.experimental.pallas import tpu_sc as plsc

N_ROWS = 1000000
N_FACTORS = 32
BATCH = 16384
NC, NS = 2, 16
NW = NC * NS
BPW = BATCH // NW
NELEM = BPW * N_FACTORS
CHUNK = 128
NCH = NELEM // CHUNK
FIRE = 32

CBLK = 32768
CSH = CBLK.bit_length() - 1
JW = CBLK // 128
NBLK = -(-N_ROWS // CBLK)
RERO = NBLK * N_FACTORS * JW


def _reblock_body(ut_ref, it_ref, uo_ref, io_ref):
    uo_ref[...] = ut_ref[...].reshape(N_FACTORS, JW, 128).reshape(
        N_FACTORS * JW, 128)
    io_ref[...] = it_ref[...].reshape(N_FACTORS, JW, 128).reshape(
        N_FACTORS * JW, 128)


def _reblock(ut_t, it_t):
    return pl.pallas_call(
        _reblock_body,
        grid=(NBLK,),
        in_specs=[
            pl.BlockSpec((N_FACTORS, CBLK), lambda i: (0, i)),
            pl.BlockSpec((N_FACTORS, CBLK), lambda i: (0, i)),
        ],
        out_specs=[
            pl.BlockSpec((N_FACTORS * JW, 128), lambda i: (i, 0)),
            pl.BlockSpec((N_FACTORS * JW, 128), lambda i: (i, 0)),
        ],
        out_shape=[
            jax.ShapeDtypeStruct((RERO, 128), jnp.float32),
            jax.ShapeDtypeStruct((RERO, 128), jnp.float32),
        ],
    )(ut_t, it_t)


def _emb_body(uid_hbm, iid_hbm, ut_hbm, it_hbm, uo_hbm, io_hbm,
              uidx_v, iidx_v, uoff_v, ioff_v, urows_v, irows_v,
              usem, isem):
    wid = lax.axis_index("s") * NC + lax.axis_index("c")
    base = wid * BPW
    pltpu.sync_copy(uid_hbm.at[pl.ds(base, BPW)], uidx_v)
    pltpu.sync_copy(iid_hbm.at[pl.ds(base, BPW)], iidx_v)

    def offs(g, _):
        for idx_v, off_v in ((uidx_v, uoff_v), (iidx_v, ioff_v)):
            u = idx_v[pl.ds(g * 16, 16)]
            b = ((u >> CSH) * (N_FACTORS * CBLK)
                 + ((u >> 7) & (JW - 1)) * 128 + (u & 127))
            for f in range(N_FACTORS):
                off_v[pl.ds(f * BPW + g * 16, 16)] = b + f * CBLK
        return 0
    lax.fori_loop(0, BPW // 16, offs, 0, unroll=False)

    def flat(v2d, c):
        return v2d.at[c // (BPW // CHUNK),
                      pl.ds((c % (BPW // CHUNK)) * CHUNK, CHUNK)]

    def gather(c0, _):
        cps = []
        for k in range(FIRE):
            c = c0 * FIRE + k
            cps.append(pltpu.async_copy(
                ut_hbm.at[uoff_v.at[pl.ds(c * CHUNK, CHUNK)]],
                flat(urows_v, c), usem))
            cps.append(pltpu.async_copy(
                it_hbm.at[ioff_v.at[pl.ds(c * CHUNK, CHUNK)]],
                flat(irows_v, c), isem))
        for cp in cps:
            cp.wait()
        return 0
    lax.fori_loop(0, NCH // FIRE, gather, 0, unroll=False)

    pltpu.sync_copy(urows_v, uo_hbm.at[:, pl.ds(base, BPW)])
    pltpu.sync_copy(irows_v, io_hbm.at[:, pl.ds(base, BPW)])


@jax.jit
def kernel(user_ids, item_ids, user_table, item_table):
    ub, ib = _reblock(user_table.T, item_table.T)
    f = pl.kernel(
        _emb_body,
        out_type=(
            jax.ShapeDtypeStruct((N_FACTORS, BATCH), jnp.float32),
            jax.ShapeDtypeStruct((N_FACTORS, BATCH), jnp.float32),
        ),
        mesh=plsc.VectorSubcoreMesh(core_axis_name="c", subcore_axis_name="s"),
        scratch_types=[
            pltpu.VMEM((BPW,), jnp.int32),
            pltpu.VMEM((BPW,), jnp.int32),
            pltpu.VMEM((NELEM,), jnp.int32),
            pltpu.VMEM((NELEM,), jnp.int32),
            pltpu.VMEM((N_FACTORS, BPW), jnp.float32),
            pltpu.VMEM((N_FACTORS, BPW), jnp.float32),
            pltpu.SemaphoreType.DMA,
            pltpu.SemaphoreType.DMA,
        ],
        compiler_params=pltpu.CompilerParams(use_tc_tiling_on_sc=False),
    )
    uo_t, io_t = f(user_ids, item_ids, ub.reshape(-1), ib.reshape(-1))
    return uo_t.T, io_t.T


# speedup vs baseline: 1.0025x; 1.0025x over previous
import jax
import jax.numpy as jnp
from jax import lax
from jax.experimental import pallas as pl
from jax.experimental.pallas import tpu as pltpu
from jax.experimental.pallas import tpu_sc as plsc

N_ROWS = 1000000
N_FACTORS = 32
BATCH = 16384
NC, NS = 2, 16
NW = NC * NS
BPW = BATCH // NW
NELEM = BPW * N_FACTORS
CHUNK = 128
NCH = NELEM // CHUNK
FIRE = 16

CBLK = 32768
CSH = CBLK.bit_length() - 1
JW = CBLK // 128
NBLK = -(-N_ROWS // CBLK)
RERO = NBLK * N_FACTORS * JW


def _reblock_body(ut_ref, it_ref, uo_ref, io_ref):
    uo_ref[...] = ut_ref[...].reshape(N_FACTORS, JW, 128).reshape(
        N_FACTORS * JW, 128)
    io_ref[...] = it_ref[...].reshape(N_FACTORS, JW, 128).reshape(
        N_FACTORS * JW, 128)


def _reblock(ut_t, it_t):
    return pl.pallas_call(
        _reblock_body,
        grid=(NBLK,),
        in_specs=[
            pl.BlockSpec((N_FACTORS, CBLK), lambda i: (0, i)),
            pl.BlockSpec((N_FACTORS, CBLK), lambda i: (0, i)),
        ],
        out_specs=[
            pl.BlockSpec((N_FACTORS * JW, 128), lambda i: (i, 0)),
            pl.BlockSpec((N_FACTORS * JW, 128), lambda i: (i, 0)),
        ],
        out_shape=[
            jax.ShapeDtypeStruct((RERO, 128), jnp.float32),
            jax.ShapeDtypeStruct((RERO, 128), jnp.float32),
        ],
    )(ut_t, it_t)


def _emb_body(uid_hbm, iid_hbm, ut_hbm, it_hbm, uo_hbm, io_hbm,
              uidx_v, iidx_v, uoff_v, ioff_v, urows_v, irows_v,
              usem, isem):
    wid = lax.axis_index("s") * NC + lax.axis_index("c")
    base = wid * BPW
    pltpu.sync_copy(uid_hbm.at[pl.ds(base, BPW)], uidx_v)
    pltpu.sync_copy(iid_hbm.at[pl.ds(base, BPW)], iidx_v)

    def offs(g, _):
        for idx_v, off_v in ((uidx_v, uoff_v), (iidx_v, ioff_v)):
            u = idx_v[pl.ds(g * 16, 16)]
            b = ((u >> CSH) * (N_FACTORS * CBLK)
                 + ((u >> 7) & (JW - 1)) * 128 + (u & 127))
            for f in range(N_FACTORS):
                off_v[pl.ds(f * BPW + g * 16, 16)] = b + f * CBLK
        return 0
    lax.fori_loop(0, BPW // 16, offs, 0, unroll=False)

    def flat(v2d, c):
        return v2d.at[c // (BPW // CHUNK),
                      pl.ds((c % (BPW // CHUNK)) * CHUNK, CHUNK)]

    def gather(c0, _):
        cps = []
        for k in range(FIRE):
            c = c0 * FIRE + k
            cps.append(pltpu.async_copy(
                ut_hbm.at[uoff_v.at[pl.ds(c * CHUNK, CHUNK)]],
                flat(urows_v, c), usem))
            cps.append(pltpu.async_copy(
                it_hbm.at[ioff_v.at[pl.ds(c * CHUNK, CHUNK)]],
                flat(irows_v, c), isem))
        for cp in cps:
            cp.wait()
        return 0
    lax.fori_loop(0, NCH // FIRE, gather, 0, unroll=False)

    pltpu.sync_copy(urows_v, uo_hbm.at[:, pl.ds(base, BPW)])
    pltpu.sync_copy(irows_v, io_hbm.at[:, pl.ds(base, BPW)])


@jax.jit
def kernel(user_ids, item_ids, user_table, item_table):
    ub, ib = _reblock(user_table.T, item_table.T)
    f = pl.kernel(
        _emb_body,
        out_type=(
            jax.ShapeDtypeStruct((N_FACTORS, BATCH), jnp.float32),
            jax.ShapeDtypeStruct((N_FACTORS, BATCH), jnp.float32),
        ),
        mesh=plsc.VectorSubcoreMesh(core_axis_name="c", subcore_axis_name="s"),
        scratch_types=[
            pltpu.VMEM((BPW,), jnp.int32),
            pltpu.VMEM((BPW,), jnp.int32),
            pltpu.VMEM((NELEM,), jnp.int32),
            pltpu.VMEM((NELEM,), jnp.int32),
            pltpu.VMEM((N_FACTORS, BPW), jnp.float32),
            pltpu.VMEM((N_FACTORS, BPW), jnp.float32),
            pltpu.SemaphoreType.DMA,
            pltpu.SemaphoreType.DMA,
        ],
        compiler_params=pltpu.CompilerParams(use_tc_tiling_on_sc=False),
    )
    uo_t, io_t = f(user_ids, item_ids, ub.reshape(-1), ib.reshape(-1))
    return uo_t.T, io_t.T
